# Initial kernel scaffold; baseline (speedup 1.0000x reference)
import jax
import jax.numpy as jnp
from jax.experimental import pallas as pl


def kernel(pts, center, feature_list, dirs1, dirs2, params):
    raise NotImplementedError("write your pallas kernel here")


# speedup vs baseline: 17.5076x; 17.5076x over previous
import functools

import jax
import jax.numpy as jnp
from jax.experimental import pallas as pl
from jax.experimental.pallas import tpu as pltpu

_F32 = jnp.float32
_HI = jax.lax.Precision.HIGHEST
_BIG_I = 2 ** 30
_MASK_F = 3.0e38


def _dotT(a, b):
    return jax.lax.dot_general(a, b, (((1,), (1,)), ((), ())),
                               precision=_HI, preferred_element_type=_F32)


def _dot(a, b):
    return jax.lax.dot_general(a, b, (((1,), (0,)), ((), ())),
                               precision=_HI, preferred_element_type=_F32)


def _pairwise_d(q, k):
    qk = _dotT(q, k)
    qn = jnp.sum(q * q, axis=1, keepdims=True)
    kn_row = _dotT(jnp.ones((1, 3), _F32), k * k)
    return (-2.0 * qk + qn) + kn_row


def _topk_min(d, k):
    n, s = d.shape
    iota = jax.lax.broadcasted_iota(jnp.int32, (n, s), 1)
    vals, idxs = [], []
    dcur = d
    for _ in range(k):
        m = jnp.min(dcur, axis=1, keepdims=True)
        ij = jnp.min(jnp.where(dcur == m, iota, _BIG_I), axis=1, keepdims=True)
        vals.append(m)
        idxs.append(ij)
        dcur = jnp.where(iota == ij, _MASK_F, dcur)
    return vals, idxs


def _onehot(idx, n, s):
    iota = jax.lax.broadcasted_iota(jnp.int32, (n, s), 1)
    return jnp.where(iota == idx, 1.0, 0.0).astype(_F32)


def _lrelu(x, slope):
    return jnp.where(x >= 0, x, slope * x)


def _fps_kernel(pts_ref, out_ref, *, niter):
    x = pts_ref[:, 0]
    y = pts_ref[:, 1]
    z = pts_ref[:, 2]
    b, r, c = x.shape
    iota = (jax.lax.broadcasted_iota(jnp.int32, (b, r, c), 1) * c
            + jax.lax.broadcasted_iota(jnp.int32, (b, r, c), 2))
    oio = jax.lax.broadcasted_iota(jnp.int32, (b, 1, niter), 2)

    def body(i, state):
        dist_min, fa, ax, ay, az = state
        msk = iota == fa
        cx = jnp.sum(jnp.where(msk, x, 0.0), axis=(1, 2), keepdims=True)
        cy = jnp.sum(jnp.where(msk, y, 0.0), axis=(1, 2), keepdims=True)
        cz = jnp.sum(jnp.where(msk, z, 0.0), axis=(1, 2), keepdims=True)
        sel = oio == i
        ax = jnp.where(sel, cx, ax)
        ay = jnp.where(sel, cy, ay)
        az = jnp.where(sel, cz, az)
        dx = x - cx
        dy = y - cy
        dz = z - cz
        d = dx * dx + dy * dy + dz * dz
        dist_min = jnp.minimum(dist_min, d)
        m = jnp.max(dist_min, axis=(1, 2), keepdims=True)
        fa = jnp.min(jnp.where(dist_min == m, iota, _BIG_I),
                     axis=(1, 2), keepdims=True)
        return dist_min, fa, ax, ay, az

    dist0 = jnp.full((b, r, c), 1e10, _F32)
    fa0 = jnp.zeros((b, 1, 1), jnp.int32)
    acc0 = jnp.zeros((b, 1, niter), _F32)
    _, _, ax, ay, az = jax.lax.fori_loop(0, niter, body,
                                         (dist0, fa0, acc0, acc0, acc0))
    out_ref[:, 0:1, :] = ax
    out_ref[:, 1:2, :] = ay
    out_ref[:, 2:3, :] = az


def _fps(pts, niter):
    B, N, _ = pts.shape
    pts_r = pts.transpose(0, 2, 1).reshape(B, 3, 8, N // 8)
    samp = pl.pallas_call(
        functools.partial(_fps_kernel, niter=niter),
        out_shape=jax.ShapeDtypeStruct((B, 3, niter), _F32),
    )(pts_r)
    return samp.transpose(0, 2, 1)


def _pfp_kernel(q_ref, k_ref, f_ref, w1a_ref, w1b_ref, b1_ref, w2_ref,
                b2_ref, out_ref):
    q = q_ref[0]
    k = k_ref[0]
    f = f_ref[0]
    n = q.shape[0]
    s = k.shape[0]
    d = _pairwise_d(q, k)
    (d1, d2, d3), (i1, i2, i3) = _topk_min(d, 3)
    r1 = 1.0 / (d1 + 1e-8)
    r2 = 1.0 / (d2 + 1e-8)
    r3 = 1.0 / (d3 + 1e-8)
    norm = (r1 + r2) + r3
    w = (jnp.where(jax.lax.broadcasted_iota(jnp.int32, (n, s), 1) == i1,
                   r1 / norm, 0.0)
         + jnp.where(jax.lax.broadcasted_iota(jnp.int32, (n, s), 1) == i2,
                     r2 / norm, 0.0)
         + jnp.where(jax.lax.broadcasted_iota(jnp.int32, (n, s), 1) == i3,
                     r3 / norm, 0.0)).astype(_F32)
    interp = _dot(w, f)
    h = _dot(q, w1a_ref[...]) + _dot(interp, w1b_ref[...]) + b1_ref[...]
    h = jnp.maximum(h, 0.0)
    o = _dot(h, w2_ref[...]) + b2_ref[...]
    out_ref[0] = jnp.maximum(o, 0.0)


def _fold_bn(w, bc, g, bb):
    sc = (1.0 / jnp.sqrt(1.0 + 1e-5)) * g
    return w * sc[:, None], bc * sc + bb


def _pfp(q_rows, k_rows, f_rows, params, prefix):
    B, n, _ = q_rows.shape
    s = k_rows.shape[1]
    C = f_rows.shape[2]
    w1, b1 = _fold_bn(params[prefix + '_conv1_w'], params[prefix + '_conv1_b'],
                      params[prefix + '_bn1_g'], params[prefix + '_bn1_b'])
    w2, b2 = _fold_bn(params[prefix + '_conv2_w'], params[prefix + '_conv2_b'],
                      params[prefix + '_bn2_g'], params[prefix + '_bn2_b'])
    w1a = w1[:, :3].T
    w1b = w1[:, 3:].T
    mid = w1.shape[0]
    wspec = lambda shp: pl.BlockSpec(shp, lambda b: (0,) * len(shp))
    return pl.pallas_call(
        _pfp_kernel,
        grid=(B,),
        in_specs=[
            pl.BlockSpec((1, n, 3), lambda b: (b, 0, 0)),
            pl.BlockSpec((1, s, 3), lambda b: (b, 0, 0)),
            pl.BlockSpec((1, s, C), lambda b: (b, 0, 0)),
            wspec((3, mid)),
            wspec((C, mid)),
            wspec((1, mid)),
            wspec((mid, C)),
            wspec((1, C)),
        ],
        out_specs=pl.BlockSpec((1, n, C), lambda b: (b, 0, 0)),
        out_shape=jax.ShapeDtypeStruct((B, n, C), _F32),
    )(q_rows, k_rows, f_rows, w1a, w1b, b1.reshape(1, mid), w2.T,
      b2.reshape(1, C))


def _group_vec(lg, scalars):
    return jnp.where(lg == 0, scalars[0],
                     jnp.where(lg == 1, scalars[1],
                               jnp.where(lg == 2, scalars[2], scalars[3])))


def _pairwise_d_vpu(q, kT, kn_row):
    qk = (q[:, 0:1] * kT[0:1, :] + q[:, 1:2] * kT[1:2, :]
          + q[:, 2:3] * kT[2:3, :])
    qn = jnp.sum(q * q, axis=1, keepdims=True)
    return (-2.0 * qk + qn) + kn_row


def _dg_layer(cq_ref, ckT_ref, p_ref, base_ref, idx_ref, cs_ref,
              gamma, beta, ch, store):
    n = cq_ref.shape[1]
    s = ckT_ref.shape[2]
    C = p_ref.shape[-1]
    nc = n // ch
    gw = C // 4
    cnt = jnp.float32(n * 4 * gw)
    lg = jax.lax.broadcasted_iota(jnp.int32, (1, C), 1) // gw

    kT = ckT_ref[0]
    kn_row = (kT[0:1, :] * kT[0:1, :] + kT[1:2, :] * kT[1:2, :]
              + kT[2:3, :] * kT[2:3, :])

    cs_ref[:, 0:C] = jnp.zeros((2, C), _F32)

    def body1(ci, _):
        q = cq_ref[0, pl.ds(ci * ch, ch), :]
        d = _pairwise_d_vpu(q, kT, kn_row)
        _, idxs = _topk_min(d, 4)
        p = p_ref[...]
        bc = base_ref[pl.ds(ci * ch, ch), 0:C]
        colsum = jnp.zeros((1, C), _F32)
        colsq = jnp.zeros((1, C), _F32)
        idx_ref[pl.ds(ci * ch, ch), 0:4] = jnp.concatenate(idxs, axis=1)
        for idx in idxs:
            t = _dot(_onehot(idx, ch, s), p) + bc
            colsum = colsum + t.sum(axis=0, keepdims=True)
            colsq = colsq + (t * t).sum(axis=0, keepdims=True)
        cs_ref[0:1, 0:C] += colsum
        cs_ref[1:2, 0:C] += colsq
        return 0

    jax.lax.fori_loop(0, nc, body1, 0)

    colsum = cs_ref[0:1, 0:C]
    colsq = cs_ref[1:2, 0:C]
    means = [jnp.sum(jnp.where(lg == g, colsum, 0.0)) / cnt for g in range(4)]
    ex2 = [jnp.sum(jnp.where(lg == g, colsq, 0.0)) / cnt for g in range(4)]
    mean_vec = _group_vec(lg, means)
    var_vec = _group_vec(lg, [ex2[g] - means[g] * means[g] for g in range(4)])
    den = jnp.sqrt(var_vec + 1e-5)

    def body2(ci, _):
        p = p_ref[...]
        bc = base_ref[pl.ds(ci * ch, ch), 0:C]
        idx4 = idx_ref[pl.ds(ci * ch, ch), 0:4]
        acc = None
        for j in range(4):
            idx = idx4[:, j:j + 1]
            t = _dot(_onehot(idx, ch, s), p) + bc
            y = _lrelu(((t - mean_vec) / den) * gamma + beta, 0.2)
            acc = y if acc is None else jnp.maximum(acc, y)
        store(ci, acc)
        return 0

    jax.lax.fori_loop(0, nc, body2, 0)


def _dg_kernel(ckT_ref, fk_ref, cq_ref, cqT_ref, fq_ref, a1_ref, bma1_ref,
               g1_ref, be1_ref, a2_ref, bma2_ref, g2_ref, be2_ref, out_ref,
               p_s, base_s, x1_s, p2_s, idx_s, cs_s, *, ch):
    p_s[...] = _dot(fk_ref[0], a1_ref[...])
    base_s[...] = _dot(fq_ref[0], bma1_ref[...])

    def store1(ci, acc):
        x1_s[pl.ds(ci * ch, ch), :] = acc

    _dg_layer(cq_ref, ckT_ref, p_s, base_s, idx_s, cs_s,
              g1_ref[...], be1_ref[...], ch, store1)

    p2_s[...] = _dot(x1_s[...], a2_ref[...])
    base_s[:, 0:384] = _dot(x1_s[...], bma2_ref[...])

    def store2(ci, acc):
        out_ref[0, pl.ds(ci * ch, ch), :] = acc

    _dg_layer(cq_ref, cqT_ref, p2_s, base_s, idx_s, cs_s,
              g2_ref[...], be2_ref[...], ch, store2)


def _dg(ck_rows, fk_rows, cq_rows, fq_rows, params, prefix):
    B, n, _ = cq_rows.shape
    s = ck_rows.shape[1]
    C = fk_rows.shape[2]
    l1 = params[prefix + '_l1_w']
    a1 = l1[:, :C].T
    bma1 = (l1[:, C:] - l1[:, :C]).T
    l2 = params[prefix + '_l2_w']
    a2 = l2[:, :512].T
    bma2 = (l2[:, 512:] - l2[:, :512]).T
    wspec = lambda shp: pl.BlockSpec(shp, lambda b: (0,) * len(shp))
    return pl.pallas_call(
        functools.partial(_dg_kernel, ch=min(n, 256)),
        grid=(B,),
        in_specs=[
            pl.BlockSpec((1, 3, s), lambda b: (b, 0, 0)),
            pl.BlockSpec((1, s, C), lambda b: (b, 0, 0)),
            pl.BlockSpec((1, n, 3), lambda b: (b, 0, 0)),
            pl.BlockSpec((1, 3, n), lambda b: (b, 0, 0)),
            pl.BlockSpec((1, n, C), lambda b: (b, 0, 0)),
            wspec((C, 512)),
            wspec((C, 512)),
            wspec((1, 512)),
            wspec((1, 512)),
            wspec((512, 384)),
            wspec((512, 384)),
            wspec((1, 384)),
            wspec((1, 384)),
        ],
        out_specs=pl.BlockSpec((1, n, 384), lambda b: (b, 0, 0)),
        out_shape=jax.ShapeDtypeStruct((B, n, 384), _F32),
        scratch_shapes=[
            pltpu.VMEM((s, 512), _F32),
            pltpu.VMEM((n, 512), _F32),
            pltpu.VMEM((n, 512), _F32),
            pltpu.VMEM((n, 384), _F32),
            pltpu.VMEM((n, 128), jnp.int32),
            pltpu.VMEM((2, 512), _F32),
        ],
    )(ck_rows.transpose(0, 2, 1), fk_rows, cq_rows,
      cq_rows.transpose(0, 2, 1), fq_rows, a1, bma1,
      params[prefix + '_gn1_g'].reshape(1, 512),
      params[prefix + '_gn1_b'].reshape(1, 512), a2, bma2,
      params[prefix + '_gn2_g'].reshape(1, 384),
      params[prefix + '_gn2_b'].reshape(1, 384))


def _head_kernel(q0_ref, k_ref, f_ref, dirs_ref, w1a_ref, w1b_ref, b1_ref,
                 w2_ref, b2_ref, m1a_ref, m1b_ref, m1bias_ref, m2_ref,
                 m2b_ref, out_ref):
    B = q0_ref.shape[0]
    for b in range(B):
        qb = q0_ref[b:b + 1, :]
        kb = k_ref[b]
        fb = f_ref[b]
        d = _pairwise_d(qb, kb)
        (d1, d2, d3), (i1, i2, i3) = _topk_min(d, 3)
        r1 = 1.0 / (d1 + 1e-8)
        r2 = 1.0 / (d2 + 1e-8)
        r3 = 1.0 / (d3 + 1e-8)
        norm = (r1 + r2) + r3
        s = kb.shape[0]
        w = (jnp.where(jax.lax.broadcasted_iota(jnp.int32, (1, s), 1) == i1,
                       r1 / norm, 0.0)
             + jnp.where(jax.lax.broadcasted_iota(jnp.int32, (1, s), 1) == i2,
                         r2 / norm, 0.0)
             + jnp.where(jax.lax.broadcasted_iota(jnp.int32, (1, s), 1) == i3,
                         r3 / norm, 0.0)).astype(_F32)
        interp = _dot(w, fb)
        h = _dot(qb, w1a_ref[...]) + _dot(interp, w1b_ref[...]) + b1_ref[...]
        h = jnp.maximum(h, 0.0)
        px = _dot(h, w2_ref[...]) + b2_ref[...]
        px = jnp.maximum(px, 0.0)
        net = (_dot(px, m1a_ref[...]) + _dot(dirs_ref[b:b + 1, :], m1b_ref[...])
               + m1bias_ref[...])
        net = _lrelu(net, 0.01)
        o = _dot(net, m2_ref[...]) + m2b_ref[...]
        out_ref[b:b + 1, :] = o


def _head(q0, k_rows, f_rows, dirs, params):
    B = q0.shape[0]
    s = k_rows.shape[1]
    C = f_rows.shape[2]
    w1, b1 = _fold_bn(params['p0_conv1_w'], params['p0_conv1_b'],
                      params['p0_bn1_g'], params['p0_bn1_b'])
    w2, b2 = _fold_bn(params['p0_conv2_w'], params['p0_conv2_b'],
                      params['p0_bn2_g'], params['p0_bn2_b'])
    mid = w1.shape[0]
    m1 = params['mlp1_w']
    out = pl.pallas_call(
        _head_kernel,
        out_shape=jax.ShapeDtypeStruct((B, 1), _F32),
    )(q0, k_rows, f_rows, dirs, w1[:, :3].T, w1[:, 3:].T, b1.reshape(1, mid),
      w2.T, b2.reshape(1, C), m1[:, :C].T, m1[:, C:].T,
      params['mlp1_b'].reshape(1, C), params['mlp2_w'].T,
      params['mlp2_b'].reshape(1, 1))
    return out[:, 0]


def kernel(pts, center, feature_list, dirs1, dirs2, params):
    samp = _fps(pts, 2048)
    lvl1 = samp
    lvl2 = samp[:, :1024]
    fl_rows = feature_list.transpose(0, 2, 1)

    f2 = _pfp(lvl2, center, fl_rows, params, 'p2')
    f1 = _pfp(lvl1, center, fl_rows, params, 'p1')
    f2 = _dg(center, fl_rows, lvl2, f2, params, 'dg2')
    f1 = _dg(lvl2, f2, lvl1, f1, params, 'dg1')

    dirs = jnp.concatenate([dirs1, dirs2], axis=1)
    return _head(pts[:, 0, :], samp, f1, dirs, params)


# speedup vs baseline: 18.7196x; 1.0692x over previous
import functools

import jax
import jax.numpy as jnp
from jax.experimental import pallas as pl
from jax.experimental.pallas import tpu as pltpu

_F32 = jnp.float32
_HI = jax.lax.Precision.HIGHEST
_BIG_I = 2 ** 30
_MASK_F = 3.0e38


def _dotT(a, b):
    return jax.lax.dot_general(a, b, (((1,), (1,)), ((), ())),
                               precision=_HI, preferred_element_type=_F32)


def _dot(a, b):
    return jax.lax.dot_general(a, b, (((1,), (0,)), ((), ())),
                               precision=_HI, preferred_element_type=_F32)


def _dotd(a, b):
    return jax.lax.dot_general(a, b, (((1,), (0,)), ((), ())),
                               preferred_element_type=_F32)


def _bn(x, g, b):
    return x / jnp.sqrt(1.0 + 1e-5) * g + b


def _pairwise_d(q, k):
    qk = _dotT(q, k)
    qn = jnp.sum(q * q, axis=1, keepdims=True)
    kn_row = _dotT(jnp.ones((1, 3), _F32), k * k)
    return (-2.0 * qk + qn) + kn_row


def _topk_min(d, k):
    n, s = d.shape
    iota = jax.lax.broadcasted_iota(jnp.int32, (n, s), 1)
    vals, idxs = [], []
    dcur = d
    for _ in range(k):
        m = jnp.min(dcur, axis=1, keepdims=True)
        ij = jnp.min(jnp.where(dcur == m, iota, _BIG_I), axis=1, keepdims=True)
        vals.append(m)
        idxs.append(ij)
        dcur = jnp.where(iota == ij, _MASK_F, dcur)
    return vals, idxs


def _onehot(idx, n, s):
    iota = jax.lax.broadcasted_iota(jnp.int32, (n, s), 1)
    return jnp.where(iota == idx, 1.0, 0.0).astype(_F32)


def _lrelu(x, slope):
    return jnp.where(x >= 0, x, slope * x)


def _fps_kernel(pts_ref, out_ref, *, niter):
    x = pts_ref[:, 0]
    y = pts_ref[:, 1]
    z = pts_ref[:, 2]
    b, r, c = x.shape
    iota = (jax.lax.broadcasted_iota(jnp.int32, (b, r, c), 1) * c
            + jax.lax.broadcasted_iota(jnp.int32, (b, r, c), 2))
    oio = jax.lax.broadcasted_iota(jnp.int32, (b, 1, niter), 2)

    def body(i, state):
        dist_min, fa, ax, ay, az = state
        msk = iota == fa
        cx = jnp.sum(jnp.where(msk, x, 0.0), axis=(1, 2), keepdims=True)
        cy = jnp.sum(jnp.where(msk, y, 0.0), axis=(1, 2), keepdims=True)
        cz = jnp.sum(jnp.where(msk, z, 0.0), axis=(1, 2), keepdims=True)
        sel = oio == i
        ax = jnp.where(sel, cx, ax)
        ay = jnp.where(sel, cy, ay)
        az = jnp.where(sel, cz, az)
        dx = x - cx
        dy = y - cy
        dz = z - cz
        d = dx * dx + dy * dy + dz * dz
        dist_min = jnp.minimum(dist_min, d)
        m = jnp.max(dist_min, axis=(1, 2), keepdims=True)
        fa = jnp.min(jnp.where(dist_min == m, iota, _BIG_I),
                     axis=(1, 2), keepdims=True)
        return dist_min, fa, ax, ay, az

    dist0 = jnp.full((b, r, c), 1e10, _F32)
    fa0 = jnp.zeros((b, 1, 1), jnp.int32)
    acc0 = jnp.zeros((b, 1, niter), _F32)
    _, _, ax, ay, az = jax.lax.fori_loop(0, niter, body,
                                         (dist0, fa0, acc0, acc0, acc0))
    out_ref[:, 0:1, :] = ax
    out_ref[:, 1:2, :] = ay
    out_ref[:, 2:3, :] = az


def _fps(pts, niter):
    B, N, _ = pts.shape
    pts_r = pts.transpose(0, 2, 1).reshape(B, 3, 8, N // 8)
    samp = pl.pallas_call(
        functools.partial(_fps_kernel, niter=niter),
        out_shape=jax.ShapeDtypeStruct((B, 3, niter), _F32),
    )(pts_r)
    return samp.transpose(0, 2, 1)


def _pfp_kernel(q_ref, k_ref, f_ref, w1a_ref, w1b_ref, cb1_ref, g1_ref,
                bb1_ref, w2_ref, cb2_ref, g2_ref, bb2_ref, out_ref):
    q = q_ref[0]
    k = k_ref[0]
    f = f_ref[0]
    n = q.shape[0]
    s = k.shape[0]
    d = _pairwise_d(q, k)
    (d1, d2, d3), (i1, i2, i3) = _topk_min(d, 3)
    r1 = 1.0 / (d1 + 1e-8)
    r2 = 1.0 / (d2 + 1e-8)
    r3 = 1.0 / (d3 + 1e-8)
    norm = (r1 + r2) + r3
    w = (jnp.where(jax.lax.broadcasted_iota(jnp.int32, (n, s), 1) == i1,
                   r1 / norm, 0.0)
         + jnp.where(jax.lax.broadcasted_iota(jnp.int32, (n, s), 1) == i2,
                     r2 / norm, 0.0)
         + jnp.where(jax.lax.broadcasted_iota(jnp.int32, (n, s), 1) == i3,
                     r3 / norm, 0.0)).astype(_F32)
    interp = _dot(w, f)
    h = _dotd(q, w1a_ref[...]) + _dotd(interp, w1b_ref[...]) + cb1_ref[...]
    h = jnp.maximum(_bn(h, g1_ref[...], bb1_ref[...]), 0.0)
    o = _dotd(h, w2_ref[...]) + cb2_ref[...]
    out_ref[0] = jnp.maximum(_bn(o, g2_ref[...], bb2_ref[...]), 0.0)


def _pfp(q_rows, k_rows, f_rows, params, prefix):
    B, n, _ = q_rows.shape
    s = k_rows.shape[1]
    C = f_rows.shape[2]
    w1 = params[prefix + '_conv1_w']
    w2 = params[prefix + '_conv2_w']
    mid = w1.shape[0]
    wspec = lambda shp: pl.BlockSpec(shp, lambda b: (0,) * len(shp))
    return pl.pallas_call(
        _pfp_kernel,
        grid=(B,),
        in_specs=[
            pl.BlockSpec((1, n, 3), lambda b: (b, 0, 0)),
            pl.BlockSpec((1, s, 3), lambda b: (b, 0, 0)),
            pl.BlockSpec((1, s, C), lambda b: (b, 0, 0)),
            wspec((3, mid)),
            wspec((C, mid)),
            wspec((1, mid)),
            wspec((1, mid)),
            wspec((1, mid)),
            wspec((mid, C)),
            wspec((1, C)),
            wspec((1, C)),
            wspec((1, C)),
        ],
        out_specs=pl.BlockSpec((1, n, C), lambda b: (b, 0, 0)),
        out_shape=jax.ShapeDtypeStruct((B, n, C), _F32),
    )(q_rows, k_rows, f_rows, w1[:, :3].T, w1[:, 3:].T,
      params[prefix + '_conv1_b'].reshape(1, mid),
      params[prefix + '_bn1_g'].reshape(1, mid),
      params[prefix + '_bn1_b'].reshape(1, mid), w2.T,
      params[prefix + '_conv2_b'].reshape(1, C),
      params[prefix + '_bn2_g'].reshape(1, C),
      params[prefix + '_bn2_b'].reshape(1, C))


def _group_vec(lg, scalars):
    return jnp.where(lg == 0, scalars[0],
                     jnp.where(lg == 1, scalars[1],
                               jnp.where(lg == 2, scalars[2], scalars[3])))


def _pairwise_d_vpu(q, kT, kn_row):
    qk = (q[:, 0:1] * kT[0:1, :] + q[:, 1:2] * kT[1:2, :]
          + q[:, 2:3] * kT[2:3, :])
    qn = jnp.sum(q * q, axis=1, keepdims=True)
    return (-2.0 * qk + qn) + kn_row


def _dg_layer(cq_ref, ckT_ref, read_src, read_q, a_ref, b_ref, idx_ref,
              cs_ref, gamma, beta, ch, cout, store):
    n = cq_ref.shape[1]
    s = ckT_ref.shape[2]
    C = cout
    nc = n // ch
    gw = C // 4
    cnt = jnp.float32(n * 4 * gw)
    lg = jax.lax.broadcasted_iota(jnp.int32, (1, C), 1) // gw

    kT = ckT_ref[0]
    kn_row = (kT[0:1, :] * kT[0:1, :] + kT[1:2, :] * kT[1:2, :]
              + kT[2:3, :] * kT[2:3, :])

    cs_ref[:, 0:C] = jnp.zeros((2, C), _F32)

    def chunk_terms(ci, idxs):
        src = read_src()
        xq = read_q(ci)
        bterm = _dotd(xq, b_ref[...])
        a = a_ref[...]
        for idx in idxs:
            g = _dot(_onehot(idx, ch, s), src)
            yield _dotd(g - xq, a) + bterm

    def body1(ci, _):
        q = cq_ref[0, pl.ds(ci * ch, ch), :]
        d = _pairwise_d_vpu(q, kT, kn_row)
        _, idxs = _topk_min(d, 4)
        colsum = jnp.zeros((1, C), _F32)
        colsq = jnp.zeros((1, C), _F32)
        idx_ref[pl.ds(ci * ch, ch), 0:4] = jnp.concatenate(idxs, axis=1)
        for t in chunk_terms(ci, idxs):
            colsum = colsum + t.sum(axis=0, keepdims=True)
            colsq = colsq + (t * t).sum(axis=0, keepdims=True)
        cs_ref[0:1, 0:C] += colsum
        cs_ref[1:2, 0:C] += colsq
        return 0

    jax.lax.fori_loop(0, nc, body1, 0)

    colsum = cs_ref[0:1, 0:C]
    colsq = cs_ref[1:2, 0:C]
    means = [jnp.sum(jnp.where(lg == g, colsum, 0.0)) / cnt for g in range(4)]
    ex2 = [jnp.sum(jnp.where(lg == g, colsq, 0.0)) / cnt for g in range(4)]
    mean_vec = _group_vec(lg, means)
    var_vec = _group_vec(lg, [ex2[g] - means[g] * means[g] for g in range(4)])
    den = jnp.sqrt(var_vec + 1e-5)

    def body2(ci, _):
        idx4 = idx_ref[pl.ds(ci * ch, ch), 0:4]
        idxs = [idx4[:, j:j + 1] for j in range(4)]
        acc = None
        for t in chunk_terms(ci, idxs):
            y = _lrelu(((t - mean_vec) / den) * gamma + beta, 0.2)
            acc = y if acc is None else jnp.maximum(acc, y)
        store(ci, acc)
        return 0

    jax.lax.fori_loop(0, nc, body2, 0)


def _dg_kernel(ckT_ref, fk_ref, cq_ref, cqT_ref, fq_ref, a1_ref, b1_ref,
               g1_ref, be1_ref, a2_ref, b2_ref, g2_ref, be2_ref, out_ref,
               x1_s, idx_s, cs_s, *, ch):
    def store1(ci, acc):
        x1_s[pl.ds(ci * ch, ch), :] = acc

    _dg_layer(cq_ref, ckT_ref, lambda: fk_ref[0],
              lambda ci: fq_ref[0, pl.ds(ci * ch, ch), :],
              a1_ref, b1_ref, idx_s, cs_s,
              g1_ref[...], be1_ref[...], ch, 512, store1)

    def store2(ci, acc):
        out_ref[0, pl.ds(ci * ch, ch), :] = acc

    _dg_layer(cq_ref, cqT_ref, lambda: x1_s[...],
              lambda ci: x1_s[pl.ds(ci * ch, ch), :],
              a2_ref, b2_ref, idx_s, cs_s,
              g2_ref[...], be2_ref[...], ch, 384, store2)


def _dg(ck_rows, fk_rows, cq_rows, fq_rows, params, prefix):
    B, n, _ = cq_rows.shape
    s = ck_rows.shape[1]
    C = fk_rows.shape[2]
    l1 = params[prefix + '_l1_w']
    a1 = l1[:, :C].T
    b1m = l1[:, C:].T
    l2 = params[prefix + '_l2_w']
    a2 = l2[:, :512].T
    b2m = l2[:, 512:].T
    wspec = lambda shp: pl.BlockSpec(shp, lambda b: (0,) * len(shp))
    return pl.pallas_call(
        functools.partial(_dg_kernel, ch=min(n, 256)),
        grid=(B,),
        in_specs=[
            pl.BlockSpec((1, 3, s), lambda b: (b, 0, 0)),
            pl.BlockSpec((1, s, C), lambda b: (b, 0, 0)),
            pl.BlockSpec((1, n, 3), lambda b: (b, 0, 0)),
            pl.BlockSpec((1, 3, n), lambda b: (b, 0, 0)),
            pl.BlockSpec((1, n, C), lambda b: (b, 0, 0)),
            wspec((C, 512)),
            wspec((C, 512)),
            wspec((1, 512)),
            wspec((1, 512)),
            wspec((512, 384)),
            wspec((512, 384)),
            wspec((1, 384)),
            wspec((1, 384)),
        ],
        out_specs=pl.BlockSpec((1, n, 384), lambda b: (b, 0, 0)),
        out_shape=jax.ShapeDtypeStruct((B, n, 384), _F32),
        scratch_shapes=[
            pltpu.VMEM((n, 512), _F32),
            pltpu.VMEM((n, 128), jnp.int32),
            pltpu.VMEM((2, 512), _F32),
        ],
    )(ck_rows.transpose(0, 2, 1), fk_rows, cq_rows,
      cq_rows.transpose(0, 2, 1), fq_rows, a1, b1m,
      params[prefix + '_gn1_g'].reshape(1, 512),
      params[prefix + '_gn1_b'].reshape(1, 512), a2, b2m,
      params[prefix + '_gn2_g'].reshape(1, 384),
      params[prefix + '_gn2_b'].reshape(1, 384))


def _head_kernel(q0_ref, k_ref, f_ref, dirs_ref, w1a_ref, w1b_ref, cb1_ref,
                 g1_ref, bb1_ref, w2_ref, cb2_ref, g2_ref, bb2_ref,
                 m1a_ref, m1b_ref, m1bias_ref, m2_ref, m2b_ref, out_ref):
    B = q0_ref.shape[0]
    for b in range(B):
        qb = q0_ref[b:b + 1, :]
        kb = k_ref[b]
        fb = f_ref[b]
        d = _pairwise_d(qb, kb)
        (d1, d2, d3), (i1, i2, i3) = _topk_min(d, 3)
        r1 = 1.0 / (d1 + 1e-8)
        r2 = 1.0 / (d2 + 1e-8)
        r3 = 1.0 / (d3 + 1e-8)
        norm = (r1 + r2) + r3
        s = kb.shape[0]
        w = (jnp.where(jax.lax.broadcasted_iota(jnp.int32, (1, s), 1) == i1,
                       r1 / norm, 0.0)
             + jnp.where(jax.lax.broadcasted_iota(jnp.int32, (1, s), 1) == i2,
                         r2 / norm, 0.0)
             + jnp.where(jax.lax.broadcasted_iota(jnp.int32, (1, s), 1) == i3,
                         r3 / norm, 0.0)).astype(_F32)
        interp = _dot(w, fb)
        h = (_dotd(qb, w1a_ref[...]) + _dotd(interp, w1b_ref[...])
             + cb1_ref[...])
        h = jnp.maximum(_bn(h, g1_ref[...], bb1_ref[...]), 0.0)
        px = _dotd(h, w2_ref[...]) + cb2_ref[...]
        px = jnp.maximum(_bn(px, g2_ref[...], bb2_ref[...]), 0.0)
        net = (_dotd(px, m1a_ref[...])
               + _dotd(dirs_ref[b:b + 1, :], m1b_ref[...]) + m1bias_ref[...])
        net = _lrelu(net, 0.01)
        o = _dotd(net, m2_ref[...]) + m2b_ref[...]
        out_ref[b:b + 1, :] = o


def _head(q0, k_rows, f_rows, dirs, params):
    B = q0.shape[0]
    C = f_rows.shape[2]
    w1 = params['p0_conv1_w']
    w2 = params['p0_conv2_w']
    mid = w1.shape[0]
    m1 = params['mlp1_w']
    out = pl.pallas_call(
        _head_kernel,
        out_shape=jax.ShapeDtypeStruct((B, 1), _F32),
    )(q0, k_rows, f_rows, dirs, w1[:, :3].T, w1[:, 3:].T,
      params['p0_conv1_b'].reshape(1, mid),
      params['p0_bn1_g'].reshape(1, mid), params['p0_bn1_b'].reshape(1, mid),
      w2.T, params['p0_conv2_b'].reshape(1, C),
      params['p0_bn2_g'].reshape(1, C), params['p0_bn2_b'].reshape(1, C),
      m1[:, :C].T, m1[:, C:].T,
      params['mlp1_b'].reshape(1, C), params['mlp2_w'].T,
      params['mlp2_b'].reshape(1, 1))
    return out[:, 0]


def kernel(pts, center, feature_list, dirs1, dirs2, params):
    samp = _fps(pts, 2048)
    lvl1 = samp
    lvl2 = samp[:, :1024]
    fl_rows = feature_list.transpose(0, 2, 1)

    f2 = _pfp(lvl2, center, fl_rows, params, 'p2')
    f1 = _pfp(lvl1, center, fl_rows, params, 'p1')
    f2 = _dg(center, fl_rows, lvl2, f2, params, 'dg2')
    f1 = _dg(lvl2, f2, lvl1, f1, params, 'dg1')

    dirs = jnp.concatenate([dirs1, dirs2], axis=1)
    return _head(pts[:, 0, :], samp, f1, dirs, params)
